# baseline (speedup 1.0000x reference)
import jax
import jax.numpy as jnp
from jax.experimental import pallas as pl
from jax.experimental.pallas import tpu as pltpu

_NUM_EXPERTS = 16
_TOP_K = 2
_BLOCK_T = 1024


def _router_body(x_ref, w_ref, scores_ref, idx_ref):
    logits = jnp.dot(x_ref[...], w_ref[...], preferred_element_type=jnp.float32)
    m = jnp.max(logits, axis=-1, keepdims=True)
    e = jnp.exp(logits - m)
    s = jnp.sum(e, axis=-1, keepdims=True)
    iota = jax.lax.broadcasted_iota(jnp.int32, logits.shape, 1)
    i1 = jnp.min(jnp.where(logits == m, iota, _NUM_EXPERTS), axis=-1,
                 keepdims=True)
    masked = jnp.where(iota == i1, -jnp.inf, logits)
    m2 = jnp.max(masked, axis=-1, keepdims=True)
    i2 = jnp.min(jnp.where(masked == m2, iota, _NUM_EXPERTS), axis=-1,
                 keepdims=True)
    v1 = 1.0 / s
    v2 = jnp.exp(m2 - m) / s
    scores_ref[...] = jnp.concatenate([v1, v2], axis=-1)
    idx_ref[...] = jnp.concatenate([i1, i2], axis=-1)


def kernel(x, router_weight):
    num_tokens, d_model = x.shape
    grid = (num_tokens // _BLOCK_T,)
    scores, indices = pl.pallas_call(
        _router_body,
        grid=grid,
        in_specs=[
            pl.BlockSpec((_BLOCK_T, d_model), lambda i: (i, 0)),
            pl.BlockSpec((d_model, _NUM_EXPERTS), lambda i: (0, 0)),
        ],
        out_specs=[
            pl.BlockSpec((_BLOCK_T, _TOP_K), lambda i: (i, 0)),
            pl.BlockSpec((_BLOCK_T, _TOP_K), lambda i: (i, 0)),
        ],
        out_shape=[
            jax.ShapeDtypeStruct((num_tokens, _TOP_K), jnp.float32),
            jax.ShapeDtypeStruct((num_tokens, _TOP_K), jnp.int32),
        ],
        compiler_params=pltpu.CompilerParams(
            dimension_semantics=("parallel",),
        ),
    )(x, router_weight)
    return scores, indices


# speedup vs baseline: 1.0496x; 1.0496x over previous
import jax
import jax.numpy as jnp
from jax.experimental import pallas as pl
from jax.experimental.pallas import tpu as pltpu

_NUM_EXPERTS = 16
_TOP_K = 2
_CHUNK = 1024
_NBUF = 4


def _topk_block(logits, scores_ref, idx_ref):
    m = jnp.max(logits, axis=-1, keepdims=True)
    e = jnp.exp(logits - m)
    s = jnp.sum(e, axis=-1, keepdims=True)
    iota = jax.lax.broadcasted_iota(jnp.int32, logits.shape, 1)
    i1 = jnp.min(jnp.where(logits == m, iota, _NUM_EXPERTS), axis=-1,
                 keepdims=True)
    masked = jnp.where(iota == i1, -jnp.inf, logits)
    m2 = jnp.max(masked, axis=-1, keepdims=True)
    i2 = jnp.min(jnp.where(masked == m2, iota, _NUM_EXPERTS), axis=-1,
                 keepdims=True)
    v1 = 1.0 / s
    v2 = jnp.exp(m2 - m) / s
    scores_ref[...] = jnp.concatenate([v1, v2], axis=-1)
    idx_ref[...] = jnp.concatenate([i1, i2], axis=-1)


def _outer_body(x_hbm, w_ref, scores_hbm, idx_hbm):
    num_tokens = x_hbm.shape[0]
    d_model = x_hbm.shape[1]

    def _inner(x_blk, scores_blk, idx_blk):
        logits = jnp.dot(x_blk[...], w_ref[...],
                         preferred_element_type=jnp.float32)
        _topk_block(logits, scores_blk, idx_blk)

    pltpu.emit_pipeline(
        _inner,
        grid=(num_tokens // _CHUNK,),
        in_specs=[
            pl.BlockSpec((_CHUNK, d_model), lambda i: (i, 0),
                         pipeline_mode=pl.Buffered(buffer_count=_NBUF)),
        ],
        out_specs=[
            pl.BlockSpec((_CHUNK, _TOP_K), lambda i: (i, 0)),
            pl.BlockSpec((_CHUNK, _TOP_K), lambda i: (i, 0)),
        ],
    )(x_hbm, scores_hbm, idx_hbm)


def kernel(x, router_weight):
    num_tokens, d_model = x.shape
    scores, indices = pl.pallas_call(
        _outer_body,
        in_specs=[
            pl.BlockSpec(memory_space=pltpu.MemorySpace.HBM),
            pl.BlockSpec(memory_space=pltpu.MemorySpace.VMEM),
        ],
        out_specs=[
            pl.BlockSpec(memory_space=pltpu.MemorySpace.HBM),
            pl.BlockSpec(memory_space=pltpu.MemorySpace.HBM),
        ],
        out_shape=[
            jax.ShapeDtypeStruct((num_tokens, _TOP_K), jnp.float32),
            jax.ShapeDtypeStruct((num_tokens, _TOP_K), jnp.int32),
        ],
    )(x, router_weight)
    return scores, indices


# speedup vs baseline: 1.0654x; 1.0150x over previous
import jax
import jax.numpy as jnp
from jax.experimental import pallas as pl
from jax.experimental.pallas import tpu as pltpu

_NUM_EXPERTS = 16
_TOP_K = 2
_CHUNK = 512
_NBUF = 8


def _topk_block(logits, scores_ref, idx_ref):
    m = jnp.max(logits, axis=-1, keepdims=True)
    e = jnp.exp(logits - m)
    s = jnp.sum(e, axis=-1, keepdims=True)
    iota = jax.lax.broadcasted_iota(jnp.int32, logits.shape, 1)
    i1 = jnp.min(jnp.where(logits == m, iota, _NUM_EXPERTS), axis=-1,
                 keepdims=True)
    masked = jnp.where(iota == i1, -jnp.inf, logits)
    m2 = jnp.max(masked, axis=-1, keepdims=True)
    i2 = jnp.min(jnp.where(masked == m2, iota, _NUM_EXPERTS), axis=-1,
                 keepdims=True)
    v1 = 1.0 / s
    v2 = jnp.exp(m2 - m) / s
    scores_ref[...] = jnp.concatenate([v1, v2], axis=-1)
    idx_ref[...] = jnp.concatenate([i1, i2], axis=-1)


def _outer_body(x_hbm, w_ref, scores_hbm, idx_hbm):
    num_tokens = x_hbm.shape[0]
    d_model = x_hbm.shape[1]

    def _inner(x_blk, scores_blk, idx_blk):
        logits = jnp.dot(x_blk[...], w_ref[...],
                         preferred_element_type=jnp.float32)
        _topk_block(logits, scores_blk, idx_blk)

    pltpu.emit_pipeline(
        _inner,
        grid=(num_tokens // _CHUNK,),
        in_specs=[
            pl.BlockSpec((_CHUNK, d_model), lambda i: (i, 0),
                         pipeline_mode=pl.Buffered(buffer_count=_NBUF)),
        ],
        out_specs=[
            pl.BlockSpec((_CHUNK, _TOP_K), lambda i: (i, 0)),
            pl.BlockSpec((_CHUNK, _TOP_K), lambda i: (i, 0)),
        ],
    )(x_hbm, scores_hbm, idx_hbm)


def kernel(x, router_weight):
    num_tokens, d_model = x.shape
    scores, indices = pl.pallas_call(
        _outer_body,
        in_specs=[
            pl.BlockSpec(memory_space=pltpu.MemorySpace.HBM),
            pl.BlockSpec(memory_space=pltpu.MemorySpace.VMEM),
        ],
        out_specs=[
            pl.BlockSpec(memory_space=pltpu.MemorySpace.HBM),
            pl.BlockSpec(memory_space=pltpu.MemorySpace.HBM),
        ],
        out_shape=[
            jax.ShapeDtypeStruct((num_tokens, _TOP_K), jnp.float32),
            jax.ShapeDtypeStruct((num_tokens, _TOP_K), jnp.int32),
        ],
    )(x, router_weight)
    return scores, indices
